# Initial kernel scaffold; baseline (speedup 1.0000x reference)
import jax
import jax.numpy as jnp
from jax.experimental import pallas as pl


def kernel(feat):
    raise NotImplementedError("write your pallas kernel here")


# speedup vs baseline: 13.8224x; 13.8224x over previous
import jax
import jax.numpy as jnp
from jax.experimental import pallas as pl

LGRAD = 27.8
LCOL = 10.0
MAXLVL = 4


def _sh_up(x):
    return jnp.concatenate([x[:1], x[:-1]], axis=0)


def _sh_down(x):
    return jnp.concatenate([x[1:], x[-1:]], axis=0)


def _sh_left(x):
    return jnp.concatenate([x[:, :1], x[:, :-1]], axis=1)


def _sh_right(x):
    return jnp.concatenate([x[:, 1:], x[:, -1:]], axis=1)


def _grad_mag(cur_hwc):
    gx = cur_hwc[:, 1:, :] - cur_hwc[:, :-1, :]
    gx = jnp.pad(gx, ((0, 0), (0, 1), (0, 0)))
    gy = cur_hwc[1:, :, :] - cur_hwc[:-1, :, :]
    gy = jnp.pad(gy, ((0, 1), (0, 0), (0, 0)))
    return jnp.sqrt(jnp.sum(gx * gx + gy * gy, axis=-1) + 1e-12)


def _one(img):
    C, H, W = img.shape
    N = H * W
    color = jnp.transpose(img, (1, 2, 0)).reshape(N, C)
    labels = jnp.arange(N, dtype=jnp.int32).reshape(H, W)
    cur = color
    for _ in range(MAXLVL):
        cur_hwc = cur.reshape(H, W, C)
        g = _grad_mag(cur_hwc)
        best_w = None
        best_l = None
        for sh in (_sh_up, _sh_down, _sh_left, _sh_right):
            cn = sh(cur_hwc)
            cd = jnp.sqrt(jnp.sum((cur_hwc - cn) ** 2, axis=-1) + 1e-12)
            gk = sh(g)
            lk = sh(labels)
            wk = LCOL * cd + LGRAD * 0.5 * (g + gk)
            wk = jnp.where(labels == lk, wk + 1e6, wk)
            if best_w is None:
                best_w, best_l = wk, lk
            else:
                take = wk < best_w
                best_l = jnp.where(take, lk, best_l)
                best_w = jnp.minimum(best_w, wk)
        p = jnp.minimum(labels.reshape(N), best_l.reshape(N))
        for _j in range(18):
            p = jnp.minimum(p, p[p])
        labels = p.reshape(H, W)
        ssum = jax.ops.segment_sum(color, p, num_segments=N)
        cnt = jax.ops.segment_sum(jnp.ones((N,), color.dtype), p, num_segments=N)
        cur = ssum[p] / jnp.clip(cnt, 1.0)[p][:, None]
    pooled = jnp.transpose(cur.reshape(H, W, C), (2, 0, 1))
    return pooled, labels


def kernel(feat):
    pooled, labels = jax.vmap(_one)(feat)
    return pooled, labels


# speedup vs baseline: 108.4718x; 7.8475x over previous
import functools

import jax
import jax.numpy as jnp
from jax import lax
from jax.experimental import pallas as pl
from jax.experimental.pallas import tpu as pltpu
from jax.experimental.pallas import tpu_sc as plsc

LGRAD = 27.8
LCOL = 10.0
MAXLVL = 4

B, C, H, W = 4, 3, 512, 512
N = H * W
NCORE = 2
NSUB = 16
IMGS_PER_CORE = B // NCORE
CH = N // NSUB
VEC = 16
MAX_DOUBLE_ROUNDS = 9


def _sh(x, k):
    if k == "u":
        return jnp.concatenate([x[:1], x[:-1]], axis=0)
    if k == "d":
        return jnp.concatenate([x[1:], x[-1:]], axis=0)
    if k == "l":
        return jnp.concatenate([x[:, :1], x[:, :-1]], axis=1)
    return jnp.concatenate([x[:, 1:], x[:, -1:]], axis=1)


def _dense_body(cur_ref, lab_ref, out_ref):
    c0 = cur_ref[0]
    c1 = cur_ref[1]
    c2 = cur_ref[2]
    lab = lab_ref[...]

    zc = jnp.zeros((H, 1), jnp.float32)
    zr = jnp.zeros((1, W), jnp.float32)

    def gsq(c):
        gx = jnp.concatenate([c[:, 1:] - c[:, :-1], zc], axis=1)
        gy = jnp.concatenate([c[1:, :] - c[:-1, :], zr], axis=0)
        return gx * gx + gy * gy

    g = jnp.sqrt((gsq(c0) + gsq(c1)) + gsq(c2) + 1e-12)

    best_w = None
    best_l = None
    for k in ("u", "d", "l", "r"):
        d0 = c0 - _sh(c0, k)
        d1 = c1 - _sh(c1, k)
        d2 = c2 - _sh(c2, k)
        cd = jnp.sqrt(((d0 * d0 + d1 * d1) + d2 * d2) + 1e-12)
        lk = _sh(lab, k)
        wk = LCOL * cd + (LGRAD * 0.5) * (g + _sh(g, k))
        wk = jnp.where(lab == lk, wk + 1e6, wk)
        if best_w is None:
            best_w, best_l = wk, lk
        else:
            take = wk < best_w
            best_l = jnp.where(take, lk, best_l)
            best_w = jnp.minimum(best_w, wk)

    out_ref[...] = jnp.minimum(lab, best_l)


def _dense(cur, labels_hw):
    return pl.pallas_call(
        _dense_body,
        grid=(B,),
        in_specs=[
            pl.BlockSpec((None, C, H, W), lambda b: (b, 0, 0, 0)),
            pl.BlockSpec((None, H, W), lambda b: (b, 0, 0)),
        ],
        out_specs=pl.BlockSpec((None, H, W), lambda b: (b, 0, 0)),
        out_shape=jax.ShapeDtypeStruct((B, H, W), jnp.int32),
    )(cur, labels_hw)


def _fill(ref, value, dtype, nwords):
    def body(i, _):
        ref[pl.ds(i * VEC, VEC)] = jnp.full((VEC,), value, dtype)
        return 0

    lax.fori_loop(0, nwords // VEC, body, 0)


SUB = 2048
NSUBBLK = CH // SUB


def _vcopy(src_ref, src_off, dst_ref, nwords):
    def body(i, _):
        dst_ref[pl.ds(i * VEC, VEC)] = src_ref[pl.ds(src_off + i * VEC, VEC)]
        return 0

    lax.fori_loop(0, nwords // VEC, body, 0)


def _sc_level_body(
    p0_hbm, color_hbm,
    lab_hbm, cur_hbm,
    pA, pB, ssum0, ssum1, ssum2, cnt,
    p_v, idx_s, g_s, c_s, d_s,
):
    cid = lax.axis_index("c")
    sid = lax.axis_index("s")
    base = sid * CH

    def jump(src_sp, dst_sp):
        for j in range(NSUBBLK):
            pltpu.sync_copy(src_sp.at[p_v.at[pl.ds(j * SUB, SUB)]], g_s)

            def body(i, _, j=j):
                a = p_v[pl.ds(j * SUB + i * VEC, VEC)]
                bv = g_s[pl.ds(i * VEC, VEC)]
                p_v[pl.ds(j * SUB + i * VEC, VEC)] = jnp.minimum(a, bv)
                return 0

            lax.fori_loop(0, SUB // VEC, body, 0)
        pltpu.sync_copy(p_v, dst_sp.at[pl.ds(base, CH)])

    for img_l in range(IMGS_PER_CORE):
        img = cid * IMGS_PER_CORE + img_l

        pltpu.sync_copy(p0_hbm.at[pl.ds(img * N + base, CH)], p_v)
        pltpu.sync_copy(p_v, pA.at[pl.ds(base, CH)])
        plsc.subcore_barrier()

        for _r in range(MAX_DOUBLE_ROUNDS):
            jump(pA, pB)
            plsc.subcore_barrier()
            jump(pB, pA)
            plsc.subcore_barrier()

        pltpu.sync_copy(p_v, lab_hbm.at[pl.ds(img * N + base, CH)])

        _fill(c_s, 0.0, jnp.float32, SUB)
        for arr in (ssum0, ssum1, ssum2, cnt):
            for j in range(NSUBBLK):
                pltpu.sync_copy(c_s, arr.at[pl.ds(base + j * SUB, SUB)])
        plsc.subcore_barrier()

        _fill(d_s, 1.0, jnp.float32, SUB)
        for j in range(NSUBBLK):
            _vcopy(p_v, j * SUB, idx_s, SUB)
            for c, arr in enumerate((ssum0, ssum1, ssum2)):
                pltpu.sync_copy(
                    color_hbm.at[pl.ds((img * C + c) * N + base + j * SUB, SUB)],
                    c_s,
                )
                pltpu.sync_copy(c_s, arr.at[idx_s], add=True)
            pltpu.sync_copy(d_s, cnt.at[idx_s], add=True)
        plsc.subcore_barrier()

        for j in range(NSUBBLK):
            _vcopy(p_v, j * SUB, idx_s, SUB)
            pltpu.sync_copy(cnt.at[idx_s], c_s)
            for c, arr in enumerate((ssum0, ssum1, ssum2)):
                pltpu.sync_copy(arr.at[idx_s], d_s)

                def div_body(i, _):
                    s = d_s[pl.ds(i * VEC, VEC)]
                    q = c_s[pl.ds(i * VEC, VEC)]
                    d_s[pl.ds(i * VEC, VEC)] = s / q
                    return 0

                lax.fori_loop(0, SUB // VEC, div_body, 0)
                pltpu.sync_copy(
                    d_s,
                    cur_hbm.at[pl.ds((img * C + c) * N + base + j * SUB, SUB)],
                )

        plsc.subcore_barrier()


def _sc_level(p0, color):
    mesh = plsc.VectorSubcoreMesh(core_axis_name="c", subcore_axis_name="s")
    return pl.kernel(
        _sc_level_body,
        out_type=(
            jax.ShapeDtypeStruct((B * N,), jnp.int32),
            jax.ShapeDtypeStruct((B * C * N,), jnp.float32),
        ),
        mesh=mesh,
        scratch_types=[
            pltpu.VMEM_SHARED((N,), jnp.int32),
            pltpu.VMEM_SHARED((N,), jnp.int32),
            pltpu.VMEM_SHARED((N,), jnp.float32),
            pltpu.VMEM_SHARED((N,), jnp.float32),
            pltpu.VMEM_SHARED((N,), jnp.float32),
            pltpu.VMEM_SHARED((N,), jnp.float32),
            pltpu.VMEM((CH,), jnp.int32),
            pltpu.VMEM((SUB,), jnp.int32),
            pltpu.VMEM((SUB,), jnp.int32),
            pltpu.VMEM((SUB,), jnp.float32),
            pltpu.VMEM((SUB,), jnp.float32),
        ],
    )(p0, color)


def kernel(feat):
    color = feat.reshape(B * C * N)
    labels = jnp.broadcast_to(
        jnp.arange(N, dtype=jnp.int32).reshape(1, H, W), (B, H, W)
    )
    cur = feat
    for _ in range(MAXLVL):
        p0 = _dense(cur, labels)
        lab_flat, cur_flat = _sc_level(p0.reshape(B * N), color)
        labels = lab_flat.reshape(B, H, W)
        cur = cur_flat.reshape(B, C, H, W)
    return cur, labels


# speedup vs baseline: 182.3405x; 1.6810x over previous
import functools

import jax
import jax.numpy as jnp
from jax import lax
from jax.experimental import pallas as pl
from jax.experimental.pallas import tpu as pltpu
from jax.experimental.pallas import tpu_sc as plsc

LGRAD = 27.8
LCOL = 10.0
MAXLVL = 4

B, C, H, W = 4, 3, 512, 512
N = H * W
NCORE = 2
NSUB = 16
IMGS_PER_CORE = B // NCORE
CH = N // NSUB
VEC = 16
MAX_DOUBLE_ROUNDS = 9


def _sh(x, k):
    if k == "u":
        return jnp.concatenate([x[:1], x[:-1]], axis=0)
    if k == "d":
        return jnp.concatenate([x[1:], x[-1:]], axis=0)
    if k == "l":
        return jnp.concatenate([x[:, :1], x[:, :-1]], axis=1)
    return jnp.concatenate([x[:, 1:], x[:, -1:]], axis=1)


def _dense_body(cur_ref, lab_ref, out_ref):
    c0 = cur_ref[0]
    c1 = cur_ref[1]
    c2 = cur_ref[2]
    lab = lab_ref[...]

    zc = jnp.zeros((H, 1), jnp.float32)
    zr = jnp.zeros((1, W), jnp.float32)

    def gsq(c):
        gx = jnp.concatenate([c[:, 1:] - c[:, :-1], zc], axis=1)
        gy = jnp.concatenate([c[1:, :] - c[:-1, :], zr], axis=0)
        return gx * gx + gy * gy

    g = jnp.sqrt((gsq(c0) + gsq(c1)) + gsq(c2) + 1e-12)

    best_w = None
    best_l = None
    for k in ("u", "d", "l", "r"):
        d0 = c0 - _sh(c0, k)
        d1 = c1 - _sh(c1, k)
        d2 = c2 - _sh(c2, k)
        cd = jnp.sqrt(((d0 * d0 + d1 * d1) + d2 * d2) + 1e-12)
        lk = _sh(lab, k)
        wk = LCOL * cd + (LGRAD * 0.5) * (g + _sh(g, k))
        wk = jnp.where(lab == lk, wk + 1e6, wk)
        if best_w is None:
            best_w, best_l = wk, lk
        else:
            take = wk < best_w
            best_l = jnp.where(take, lk, best_l)
            best_w = jnp.minimum(best_w, wk)

    out_ref[...] = jnp.minimum(lab, best_l)


def _dense(cur, labels_hw):
    return pl.pallas_call(
        _dense_body,
        grid=(B,),
        in_specs=[
            pl.BlockSpec((None, C, H, W), lambda b: (b, 0, 0, 0)),
            pl.BlockSpec((None, H, W), lambda b: (b, 0, 0)),
        ],
        out_specs=pl.BlockSpec((None, H, W), lambda b: (b, 0, 0)),
        out_shape=jax.ShapeDtypeStruct((B, H, W), jnp.int32),
    )(cur, labels_hw)


def _fill(ref, value, dtype, nwords):
    def body(i, _):
        ref[pl.ds(i * VEC, VEC)] = jnp.full((VEC,), value, dtype)
        return 0

    lax.fori_loop(0, nwords // VEC, body, 0)


SUB = 2048
NSUBBLK = CH // SUB
GSUB = 2048
MAX_JUMPS = 2 * MAX_DOUBLE_ROUNDS


def _vcopy(src_ref, src_off, dst_ref, nwords):
    def body(i, _):
        dst_ref[pl.ds(i * VEC, VEC)] = src_ref[pl.ds(src_off + i * VEC, VEC)]
        return 0

    lax.fori_loop(0, nwords // VEC, body, 0)


def _sc_level_body(
    p0_hbm, color_hbm,
    lab_hbm, cur_hbm,
    pA, pB, ssum0, ssum1, ssum2, cnt, flags,
    p_v, idx_s, g_s, c_s, d_s, f_v, ff_v,
):
    cid = lax.axis_index("c")
    sid = lax.axis_index("s")
    base = sid * CH

    def jump_active(src_sp, dst_sp):
        acc = jnp.zeros((VEC,), jnp.int32)
        for j in range(CH // GSUB):
            pltpu.sync_copy(src_sp.at[p_v.at[pl.ds(j * GSUB, GSUB)]], g_s)

            def body(i, a, j=j):
                av = p_v[pl.ds(j * GSUB + i * VEC, VEC)]
                bv = g_s[pl.ds(i * VEC, VEC)]
                nw = jnp.minimum(av, bv)
                p_v[pl.ds(j * GSUB + i * VEC, VEC)] = nw
                return a | (av ^ nw)

            acc = lax.fori_loop(0, GSUB // VEC, body, acc)
        pltpu.sync_copy(p_v, dst_sp.at[pl.ds(base, CH)])
        f_v[...] = acc
        pltpu.sync_copy(f_v, flags.at[pl.ds(sid * VEC, VEC)])

    for img_l in range(IMGS_PER_CORE):
        img = cid * IMGS_PER_CORE + img_l

        pltpu.sync_copy(p0_hbm.at[pl.ds(img * N + base, CH)], p_v)
        pltpu.sync_copy(p_v, pA.at[pl.ds(base, CH)])
        plsc.subcore_barrier()

        def dround(_, done):
            active = done == 0

            @pl.when(active)
            def _():
                jump_active(pA, pB)

            plsc.subcore_barrier()

            @pl.when(active)
            def _():
                jump_active(pB, pA)

            plsc.subcore_barrier()

            @pl.when(active)
            def _():
                pltpu.sync_copy(flags, ff_v)

            comb = ff_v[pl.ds(0, VEC)]
            for s in range(1, NSUB):
                comb = comb | ff_v[pl.ds(s * VEC, VEC)]
            orv = comb[0]
            for i in range(1, VEC):
                orv = orv | comb[i]
            done = jnp.where(orv == 0, 1, done).astype(jnp.int32)
            plsc.subcore_barrier()
            return done

        lax.fori_loop(0, MAX_DOUBLE_ROUNDS, dround, jnp.int32(0))

        pltpu.sync_copy(p_v, lab_hbm.at[pl.ds(img * N + base, CH)])

        _fill(c_s, 0.0, jnp.float32, SUB)
        for arr in (ssum0, ssum1, ssum2, cnt):
            for j in range(NSUBBLK):
                pltpu.sync_copy(c_s, arr.at[pl.ds(base + j * SUB, SUB)])
        plsc.subcore_barrier()

        _fill(d_s, 1.0, jnp.float32, SUB)
        for j in range(NSUBBLK):
            _vcopy(p_v, j * SUB, idx_s, SUB)
            for c, arr in enumerate((ssum0, ssum1, ssum2)):
                pltpu.sync_copy(
                    color_hbm.at[pl.ds((img * C + c) * N + base + j * SUB, SUB)],
                    c_s,
                )
                pltpu.sync_copy(c_s, arr.at[idx_s], add=True)
            pltpu.sync_copy(d_s, cnt.at[idx_s], add=True)
        plsc.subcore_barrier()

        for j in range(NSUBBLK):
            _vcopy(p_v, j * SUB, idx_s, SUB)
            pltpu.sync_copy(cnt.at[idx_s], c_s)
            for c, arr in enumerate((ssum0, ssum1, ssum2)):
                pltpu.sync_copy(arr.at[idx_s], d_s)

                def div_body(i, _):
                    s = d_s[pl.ds(i * VEC, VEC)]
                    q = c_s[pl.ds(i * VEC, VEC)]
                    d_s[pl.ds(i * VEC, VEC)] = s / q
                    return 0

                lax.fori_loop(0, SUB // VEC, div_body, 0)
                pltpu.sync_copy(
                    d_s,
                    cur_hbm.at[pl.ds((img * C + c) * N + base + j * SUB, SUB)],
                )

        plsc.subcore_barrier()


def _sc_level(p0, color):
    mesh = plsc.VectorSubcoreMesh(core_axis_name="c", subcore_axis_name="s")
    return pl.kernel(
        _sc_level_body,
        out_type=(
            jax.ShapeDtypeStruct((B * N,), jnp.int32),
            jax.ShapeDtypeStruct((B * C * N,), jnp.float32),
        ),
        mesh=mesh,
        scratch_types=[
            pltpu.VMEM_SHARED((N,), jnp.int32),
            pltpu.VMEM_SHARED((N,), jnp.int32),
            pltpu.VMEM_SHARED((N,), jnp.float32),
            pltpu.VMEM_SHARED((N,), jnp.float32),
            pltpu.VMEM_SHARED((N,), jnp.float32),
            pltpu.VMEM_SHARED((N,), jnp.float32),
            pltpu.VMEM_SHARED((NSUB * VEC,), jnp.int32),
            pltpu.VMEM((CH,), jnp.int32),
            pltpu.VMEM((SUB,), jnp.int32),
            pltpu.VMEM((GSUB,), jnp.int32),
            pltpu.VMEM((SUB,), jnp.float32),
            pltpu.VMEM((SUB,), jnp.float32),
            pltpu.VMEM((VEC,), jnp.int32),
            pltpu.VMEM((NSUB * VEC,), jnp.int32),
        ],
    )(p0, color)


def kernel(feat):
    color = feat.reshape(B * C * N)
    labels = jnp.broadcast_to(
        jnp.arange(N, dtype=jnp.int32).reshape(1, H, W), (B, H, W)
    )
    cur = feat
    for _ in range(MAXLVL):
        p0 = _dense(cur, labels)
        lab_flat, cur_flat = _sc_level(p0.reshape(B * N), color)
        labels = lab_flat.reshape(B, H, W)
        cur = cur_flat.reshape(B, C, H, W)
    return cur, labels


# speedup vs baseline: 183.1250x; 1.0043x over previous
import functools

import jax
import jax.numpy as jnp
from jax import lax
from jax.experimental import pallas as pl
from jax.experimental.pallas import tpu as pltpu
from jax.experimental.pallas import tpu_sc as plsc

LGRAD = 27.8
LCOL = 10.0
MAXLVL = 4

B, C, H, W = 4, 3, 512, 512
N = H * W
NCORE = 2
NSUB = 16
IMGS_PER_CORE = B // NCORE
CH = N // NSUB
VEC = 16
MAX_DOUBLE_ROUNDS = 9


def _sh(x, k):
    if k == "u":
        return jnp.concatenate([x[:1], x[:-1]], axis=0)
    if k == "d":
        return jnp.concatenate([x[1:], x[-1:]], axis=0)
    if k == "l":
        return jnp.concatenate([x[:, :1], x[:, :-1]], axis=1)
    return jnp.concatenate([x[:, 1:], x[:, -1:]], axis=1)


def _dense_body(cur_ref, lab_ref, out_ref):
    c0 = cur_ref[0]
    c1 = cur_ref[1]
    c2 = cur_ref[2]
    lab = lab_ref[...]

    zc = jnp.zeros((H, 1), jnp.float32)
    zr = jnp.zeros((1, W), jnp.float32)

    def gsq(c):
        gx = jnp.concatenate([c[:, 1:] - c[:, :-1], zc], axis=1)
        gy = jnp.concatenate([c[1:, :] - c[:-1, :], zr], axis=0)
        return gx * gx + gy * gy

    g = jnp.sqrt((gsq(c0) + gsq(c1)) + gsq(c2) + 1e-12)

    best_w = None
    best_l = None
    for k in ("u", "d", "l", "r"):
        d0 = c0 - _sh(c0, k)
        d1 = c1 - _sh(c1, k)
        d2 = c2 - _sh(c2, k)
        cd = jnp.sqrt(((d0 * d0 + d1 * d1) + d2 * d2) + 1e-12)
        lk = _sh(lab, k)
        wk = LCOL * cd + (LGRAD * 0.5) * (g + _sh(g, k))
        wk = jnp.where(lab == lk, wk + 1e6, wk)
        if best_w is None:
            best_w, best_l = wk, lk
        else:
            take = wk < best_w
            best_l = jnp.where(take, lk, best_l)
            best_w = jnp.minimum(best_w, wk)

    out_ref[...] = jnp.minimum(lab, best_l)


def _dense(cur, labels_hw):
    return pl.pallas_call(
        _dense_body,
        grid=(B,),
        in_specs=[
            pl.BlockSpec((None, C, H, W), lambda b: (b, 0, 0, 0)),
            pl.BlockSpec((None, H, W), lambda b: (b, 0, 0)),
        ],
        out_specs=pl.BlockSpec((None, H, W), lambda b: (b, 0, 0)),
        out_shape=jax.ShapeDtypeStruct((B, H, W), jnp.int32),
    )(cur, labels_hw)


def _fill(ref, value, dtype, nwords):
    def body(i, _):
        ref[pl.ds(i * VEC, VEC)] = jnp.full((VEC,), value, dtype)
        return 0

    lax.fori_loop(0, nwords // VEC, body, 0)


SUB = 2048
NSUBBLK = CH // SUB
GSUB = 8192
MAX_JUMPS = 2 * MAX_DOUBLE_ROUNDS


def _vcopy(src_ref, src_off, dst_ref, nwords):
    def body(i, _):
        dst_ref[pl.ds(i * VEC, VEC)] = src_ref[pl.ds(src_off + i * VEC, VEC)]
        return 0

    lax.fori_loop(0, nwords // VEC, body, 0)


def _sc_level_body(
    p0_hbm, color_hbm,
    lab_hbm, cur_hbm,
    pA, pB, ssum0, ssum1, ssum2, cnt, flags,
    p_v, idx_s, g_s, c_s, d_s, f_v, ff_v,
):
    cid = lax.axis_index("c")
    sid = lax.axis_index("s")
    base = sid * CH

    def jump_active(src_sp, dst_sp):
        acc = jnp.zeros((VEC,), jnp.int32)
        for j in range(CH // GSUB):
            pltpu.sync_copy(src_sp.at[p_v.at[pl.ds(j * GSUB, GSUB)]], g_s)

            def body(i, a, j=j):
                av = p_v[pl.ds(j * GSUB + i * VEC, VEC)]
                bv = g_s[pl.ds(i * VEC, VEC)]
                nw = jnp.minimum(av, bv)
                p_v[pl.ds(j * GSUB + i * VEC, VEC)] = nw
                return a | (av ^ nw)

            acc = lax.fori_loop(0, GSUB // VEC, body, acc)
        pltpu.sync_copy(p_v, dst_sp.at[pl.ds(base, CH)])
        f_v[...] = acc
        pltpu.sync_copy(f_v, flags.at[pl.ds(sid * VEC, VEC)])

    for img_l in range(IMGS_PER_CORE):
        img = cid * IMGS_PER_CORE + img_l

        pltpu.sync_copy(p0_hbm.at[pl.ds(img * N + base, CH)], p_v)
        pltpu.sync_copy(p_v, pA.at[pl.ds(base, CH)])
        plsc.subcore_barrier()

        def dround(_, done):
            active = done == 0

            @pl.when(active)
            def _():
                jump_active(pA, pB)

            plsc.subcore_barrier()

            @pl.when(active)
            def _():
                jump_active(pB, pA)

            plsc.subcore_barrier()

            @pl.when(active)
            def _():
                pltpu.sync_copy(flags, ff_v)

            comb = ff_v[pl.ds(0, VEC)]
            for s in range(1, NSUB):
                comb = comb | ff_v[pl.ds(s * VEC, VEC)]
            orv = comb[0]
            for i in range(1, VEC):
                orv = orv | comb[i]
            done = jnp.where(orv == 0, 1, done).astype(jnp.int32)
            plsc.subcore_barrier()
            return done

        lax.fori_loop(0, MAX_DOUBLE_ROUNDS, dround, jnp.int32(0))

        pltpu.sync_copy(p_v, lab_hbm.at[pl.ds(img * N + base, CH)])

        _fill(c_s, 0.0, jnp.float32, SUB)
        for arr in (ssum0, ssum1, ssum2, cnt):
            for j in range(NSUBBLK):
                pltpu.sync_copy(c_s, arr.at[pl.ds(base + j * SUB, SUB)])
        plsc.subcore_barrier()

        _fill(d_s, 1.0, jnp.float32, SUB)
        for j in range(NSUBBLK):
            _vcopy(p_v, j * SUB, idx_s, SUB)
            for c, arr in enumerate((ssum0, ssum1, ssum2)):
                pltpu.sync_copy(
                    color_hbm.at[pl.ds((img * C + c) * N + base + j * SUB, SUB)],
                    c_s,
                )
                pltpu.sync_copy(c_s, arr.at[idx_s], add=True)
            pltpu.sync_copy(d_s, cnt.at[idx_s], add=True)
        plsc.subcore_barrier()

        for j in range(NSUBBLK):
            _vcopy(p_v, j * SUB, idx_s, SUB)
            pltpu.sync_copy(cnt.at[idx_s], c_s)
            for c, arr in enumerate((ssum0, ssum1, ssum2)):
                pltpu.sync_copy(arr.at[idx_s], d_s)

                def div_body(i, _):
                    s = d_s[pl.ds(i * VEC, VEC)]
                    q = c_s[pl.ds(i * VEC, VEC)]
                    d_s[pl.ds(i * VEC, VEC)] = s / q
                    return 0

                lax.fori_loop(0, SUB // VEC, div_body, 0)
                pltpu.sync_copy(
                    d_s,
                    cur_hbm.at[pl.ds((img * C + c) * N + base + j * SUB, SUB)],
                )

        plsc.subcore_barrier()


def _sc_level(p0, color):
    mesh = plsc.VectorSubcoreMesh(core_axis_name="c", subcore_axis_name="s")
    return pl.kernel(
        _sc_level_body,
        out_type=(
            jax.ShapeDtypeStruct((B * N,), jnp.int32),
            jax.ShapeDtypeStruct((B * C * N,), jnp.float32),
        ),
        mesh=mesh,
        scratch_types=[
            pltpu.VMEM_SHARED((N,), jnp.int32),
            pltpu.VMEM_SHARED((N,), jnp.int32),
            pltpu.VMEM_SHARED((N,), jnp.float32),
            pltpu.VMEM_SHARED((N,), jnp.float32),
            pltpu.VMEM_SHARED((N,), jnp.float32),
            pltpu.VMEM_SHARED((N,), jnp.float32),
            pltpu.VMEM_SHARED((NSUB * VEC,), jnp.int32),
            pltpu.VMEM((CH,), jnp.int32),
            pltpu.VMEM((SUB,), jnp.int32),
            pltpu.VMEM((GSUB,), jnp.int32),
            pltpu.VMEM((SUB,), jnp.float32),
            pltpu.VMEM((SUB,), jnp.float32),
            pltpu.VMEM((VEC,), jnp.int32),
            pltpu.VMEM((NSUB * VEC,), jnp.int32),
        ],
    )(p0, color)


def kernel(feat):
    color = feat.reshape(B * C * N)
    labels = jnp.broadcast_to(
        jnp.arange(N, dtype=jnp.int32).reshape(1, H, W), (B, H, W)
    )
    cur = feat
    for _ in range(MAXLVL):
        p0 = _dense(cur, labels)
        lab_flat, cur_flat = _sc_level(p0.reshape(B * N), color)
        labels = lab_flat.reshape(B, H, W)
        cur = cur_flat.reshape(B, C, H, W)
    return cur, labels
